# baseline (speedup 1.0000x reference)
import jax
import jax.numpy as jnp
from jax.experimental import pallas as pl

_N = 10000
_E = 320000
_D = 128


def _dot_body(u_ref, e_ref, o_ref):
    o_ref[:] = jnp.sum(u_ref[:] * e_ref[:], axis=-1)


def kernel(edge_index, edge_label_index, n_items, U_0):
    m = U_0.shape[0]
    n = _N
    src = edge_index[0]
    dst = edge_index[1] % n_items
    deg_u = jnp.zeros((m,), jnp.float32).at[src].add(1.0)
    deg_i = jnp.zeros((n,), jnp.float32).at[dst].add(1.0)
    inv_sqrt_u = deg_u ** -0.5
    inv_sqrt_i = deg_i ** -0.5
    E_1 = jax.ops.segment_sum(U_0[src], dst, num_segments=n) * inv_sqrt_i[:, None]
    U_2 = jax.ops.segment_sum(E_1[dst], src, num_segments=m) * inv_sqrt_u[:, None]
    E_3 = jax.ops.segment_sum(U_2[src], dst, num_segments=n) * inv_sqrt_i[:, None]
    E = 0.5 * (E_1 + E_3)
    U = 0.5 * (U_0 + U_2)
    s = U[edge_label_index[0]].reshape(5000, 64, _D)
    d = E[edge_label_index[1]].reshape(5000, 64, _D)
    out = pl.pallas_call(
        _dot_body,
        out_shape=jax.ShapeDtypeStruct((5000, 64), jnp.float32),
        grid=(25,),
        in_specs=[
            pl.BlockSpec((200, 64, _D), lambda i: (i, 0, 0)),
            pl.BlockSpec((200, 64, _D), lambda i: (i, 0, 0)),
        ],
        out_specs=pl.BlockSpec((200, 64), lambda i: (i, 0)),
    )(s, d)
    return out.reshape(_E)


# speedup vs baseline: 3.9696x; 3.9696x over previous
import functools

import jax
import jax.numpy as jnp
from jax import lax
from jax.experimental import pallas as pl
from jax.experimental.pallas import tpu as pltpu
from jax.experimental.pallas import tpu_sc as plsc

_N = 10000
_E = 320000
_D = 128
_NC = 2
_NS = 16
_NW = _NC * _NS
_K = 128
_CHUNKS = _E // _K
_TRIPS = -(-_CHUNKS // _NW)
_RPT8 = 624
_NPAD = 10240
_DPT = _NPAD // _NS
_BR = 1000

_mesh = plsc.VectorSubcoreMesh(core_axis_name="c", subcore_axis_name="s")

_Z16 = functools.partial(jnp.zeros, (16,), jnp.float32)


def _zero_rows(rows_v):
    z = _Z16()

    def zrow(r, carry):
        for c8 in range(8):
            rows_v[r, pl.ds(c8 * 16, 16)] = z
        return carry

    lax.fori_loop(0, _K, zrow, 0)


def _zero_acc(rows_v, acc_sh, sid):
    base_r = sid * _RPT8
    for k in range(4):
        pltpu.sync_copy(rows_v, acc_sh.at[pl.ds(base_r + k * 128, 128)])
    pltpu.sync_copy(rows_v.at[pl.ds(0, 112)],
                    acc_sh.at[pl.ds(base_r + 512, 112)])

    @pl.when(sid == _NS - 1)
    def _():
        pltpu.sync_copy(rows_v.at[pl.ds(0, 16)],
                        acc_sh.at[pl.ds(_NS * _RPT8, 16)])


def _write_acc(acc_sh, out, cid, sid):
    pltpu.sync_copy(acc_sh.at[pl.ds(sid * _RPT8, _RPT8)],
                    out.at[pl.ds(cid * _N + sid * _RPT8, _RPT8)])

    @pl.when(sid == _NS - 1)
    def _():
        pltpu.sync_copy(acc_sh.at[pl.ds(_NS * _RPT8, 16)],
                        out.at[pl.ds(cid * _N + _NS * _RPT8, 16)])


def _segsum_deg_body(table, gidx, sidx, out, dgu, dgi,
                     gidx_v, sidx_v, rows_v, ones_v,
                     acc_sh, degu_sh, degi_sh, sem):
    cid = lax.axis_index("c")
    sid = lax.axis_index("s")
    wid = sid * _NC + cid

    _zero_rows(rows_v)
    _zero_acc(rows_v, acc_sh, sid)
    for k in range(5):
        off = sid * _DPT + k * 128
        pltpu.sync_copy(rows_v.at[0], degu_sh.at[pl.ds(off, 128)])
        pltpu.sync_copy(rows_v.at[0], degi_sh.at[pl.ds(off, 128)])
    one = jnp.ones((16,), jnp.float32)
    for c8 in range(8):
        ones_v[pl.ds(c8 * 16, 16)] = one
    plsc.subcore_barrier()

    def trip(i, carry):
        c = wid + i * _NW

        @pl.when(c < _CHUNKS)
        def _():
            base = c * _K
            pltpu.sync_copy(gidx.at[pl.ds(base, _K)], gidx_v)
            pltpu.sync_copy(sidx.at[pl.ds(base, _K)], sidx_v)
            pltpu.async_copy(table.at[gidx_v], rows_v, sem).wait()
            pltpu.sync_copy(rows_v, acc_sh.at[sidx_v], add=True)
            pltpu.sync_copy(ones_v, degu_sh.at[gidx_v], add=True)
            pltpu.sync_copy(ones_v, degi_sh.at[sidx_v], add=True)

        return carry

    lax.fori_loop(0, _TRIPS, trip, 0)
    plsc.subcore_barrier()

    _write_acc(acc_sh, out, cid, sid)
    db = cid * _NPAD + sid * _DPT
    pltpu.sync_copy(degu_sh.at[pl.ds(sid * _DPT, _DPT)],
                    dgu.at[pl.ds(db, _DPT)])
    pltpu.sync_copy(degi_sh.at[pl.ds(sid * _DPT, _DPT)],
                    dgi.at[pl.ds(db, _DPT)])


def _segsum_body(table, gidx, sidx, out,
                 gidx_v, sidx_v, rows_v, acc_sh, sem):
    cid = lax.axis_index("c")
    sid = lax.axis_index("s")
    wid = sid * _NC + cid

    _zero_rows(rows_v)
    _zero_acc(rows_v, acc_sh, sid)
    plsc.subcore_barrier()

    def trip(i, carry):
        c = wid + i * _NW

        @pl.when(c < _CHUNKS)
        def _():
            base = c * _K
            pltpu.sync_copy(gidx.at[pl.ds(base, _K)], gidx_v)
            pltpu.sync_copy(sidx.at[pl.ds(base, _K)], sidx_v)
            pltpu.async_copy(table.at[gidx_v], rows_v, sem).wait()
            pltpu.sync_copy(rows_v, acc_sh.at[sidx_v], add=True)

        return carry

    lax.fori_loop(0, _TRIPS, trip, 0)
    plsc.subcore_barrier()

    _write_acc(acc_sh, out, cid, sid)


def _dot_body(u_hbm, e_hbm, aidx, bidx, out,
              aidx_v, bidx_v, urows_v, erows_v, out_v, sem_u, sem_e):
    cid = lax.axis_index("c")
    sid = lax.axis_index("s")
    wid = sid * _NC + cid
    rowiota = lax.iota(jnp.int32, 16)

    def trip(i, carry):
        c = wid + i * _NW

        @pl.when(c < _CHUNKS)
        def _():
            base = c * _K
            pltpu.sync_copy(aidx.at[pl.ds(base, _K)], aidx_v)
            pltpu.sync_copy(bidx.at[pl.ds(base, _K)], bidx_v)
            cu = pltpu.async_copy(u_hbm.at[aidx_v], urows_v, sem_u)
            ce = pltpu.async_copy(e_hbm.at[bidx_v], erows_v, sem_e)
            cu.wait()
            ce.wait()

            def gloop(g, carry):
                gbase = g * 16
                out_vec = _Z16()
                for e in range(16):
                    row = gbase + e
                    acc = _Z16()
                    for k8 in range(8):
                        uc = urows_v[row, pl.ds(k8 * 16, 16)]
                        ec = erows_v[row, pl.ds(k8 * 16, 16)]
                        acc = acc + uc * ec
                    for sh in (8, 4, 2, 1):
                        pidx = (rowiota + sh) & 15
                        acc = acc + acc.at[pidx].get(mode="promise_in_bounds")
                    out_vec = jnp.where(rowiota == e, acc, out_vec)
                out_v[pl.ds(gbase, 16)] = out_vec
                return carry

            lax.fori_loop(0, 8, gloop, 0)
            pltpu.sync_copy(out_v, out.at[pl.ds(base, _K)])

        return carry

    lax.fori_loop(0, _TRIPS, trip, 0)


_segsum_deg = pl.kernel(
    _segsum_deg_body,
    out_type=(jax.ShapeDtypeStruct((_NC * _N, _D), jnp.float32),
              jax.ShapeDtypeStruct((_NC * _NPAD,), jnp.float32),
              jax.ShapeDtypeStruct((_NC * _NPAD,), jnp.float32)),
    mesh=_mesh,
    scratch_types=(pltpu.VMEM((_K,), jnp.int32),
                   pltpu.VMEM((_K,), jnp.int32),
                   pltpu.VMEM((_K, _D), jnp.float32),
                   pltpu.VMEM((_K,), jnp.float32),
                   pltpu.VMEM_SHARED((_N, _D), jnp.float32),
                   pltpu.VMEM_SHARED((_NPAD,), jnp.float32),
                   pltpu.VMEM_SHARED((_NPAD,), jnp.float32),
                   pltpu.SemaphoreType.DMA),
)

_segsum = pl.kernel(
    _segsum_body,
    out_type=jax.ShapeDtypeStruct((_NC * _N, _D), jnp.float32),
    mesh=_mesh,
    scratch_types=(pltpu.VMEM((_K,), jnp.int32),
                   pltpu.VMEM((_K,), jnp.int32),
                   pltpu.VMEM((_K, _D), jnp.float32),
                   pltpu.VMEM_SHARED((_N, _D), jnp.float32),
                   pltpu.SemaphoreType.DMA),
)

_dot = pl.kernel(
    _dot_body,
    out_type=jax.ShapeDtypeStruct((_E,), jnp.float32),
    mesh=_mesh,
    scratch_types=(pltpu.VMEM((_K,), jnp.int32),
                   pltpu.VMEM((_K,), jnp.int32),
                   pltpu.VMEM((_K, _D), jnp.float32),
                   pltpu.VMEM((_K, _D), jnp.float32),
                   pltpu.VMEM((_K,), jnp.float32),
                   pltpu.SemaphoreType.DMA,
                   pltpu.SemaphoreType.DMA),
)


def _norm_items_fn(p_ref, deg_ref, o_ref):
    inv = lax.rsqrt(deg_ref[0] + deg_ref[1])
    o_ref[:] = (p_ref[0] + p_ref[1]) * inv


def _norm_users_fn(q_ref, deg_ref, u0_ref, u2_ref, u_ref):
    inv = lax.rsqrt(deg_ref[0] + deg_ref[1])
    u2 = (q_ref[0] + q_ref[1]) * inv
    u2_ref[:] = u2
    u_ref[:] = 0.5 * (u0_ref[:] + u2)


def _norm_items2_fn(r_ref, deg_ref, e1_ref, e_ref):
    inv = lax.rsqrt(deg_ref[0] + deg_ref[1])
    e_ref[:] = 0.5 * (e1_ref[:] + (r_ref[0] + r_ref[1]) * inv)


_p_spec = pl.BlockSpec((2, _BR, _D), lambda i: (0, i, 0))
_deg_spec = pl.BlockSpec((2, _BR, 1), lambda i: (0, i, 0))
_row_spec = pl.BlockSpec((_BR, _D), lambda i: (i, 0))

_norm_items = pl.pallas_call(
    _norm_items_fn,
    out_shape=jax.ShapeDtypeStruct((_N, _D), jnp.float32),
    grid=(_N // _BR,),
    in_specs=[_p_spec, _deg_spec],
    out_specs=_row_spec,
)

_norm_users = pl.pallas_call(
    _norm_users_fn,
    out_shape=(jax.ShapeDtypeStruct((_N, _D), jnp.float32),
               jax.ShapeDtypeStruct((_N, _D), jnp.float32)),
    grid=(_N // _BR,),
    in_specs=[_p_spec, _deg_spec, _row_spec],
    out_specs=(_row_spec, _row_spec),
)

_norm_items2 = pl.pallas_call(
    _norm_items2_fn,
    out_shape=jax.ShapeDtypeStruct((_N, _D), jnp.float32),
    grid=(_N // _BR,),
    in_specs=[_p_spec, _deg_spec, _row_spec],
    out_specs=_row_spec,
)


def kernel(edge_index, edge_label_index, n_items, U_0):
    src = edge_index[0]
    dst = (edge_index[1] % n_items).astype(jnp.int32)
    a = edge_label_index[0]
    b = edge_label_index[1]

    p1, dgu, dgi = _segsum_deg(U_0, src, dst)
    dgu = dgu.reshape(_NC, _NPAD)[:, :_N, None]
    dgi = dgi.reshape(_NC, _NPAD)[:, :_N, None]
    E1 = _norm_items(p1.reshape(_NC, _N, _D), dgi)
    q = _segsum(E1, dst, src)
    U2, U = _norm_users(q.reshape(_NC, _N, _D), dgu, U_0)
    r = _segsum(U2, src, dst)
    E = _norm_items2(r.reshape(_NC, _N, _D), dgi, E1)
    return _dot(U, E, a, b)


# speedup vs baseline: 5.7556x; 1.4499x over previous
import functools

import jax
import jax.numpy as jnp
from jax import lax
from jax.experimental import pallas as pl
from jax.experimental.pallas import tpu as pltpu
from jax.experimental.pallas import tpu_sc as plsc

_N = 10000
_E = 320000
_D = 128
_NC = 2
_NS = 16
_NW = _NC * _NS
_K = 128
_CHUNKS = _E // _K
_TRIPS = -(-_CHUNKS // _NW)
_RPT8 = 624
_NPAD = 10240
_DPT = _NPAD // _NS
_BR = 1000

_mesh = plsc.VectorSubcoreMesh(core_axis_name="c", subcore_axis_name="s")

_Z16 = functools.partial(jnp.zeros, (16,), jnp.float32)


def _zero_rows(rows_v):
    z = _Z16()

    def zrow(r, carry):
        for c8 in range(8):
            rows_v[r, pl.ds(c8 * 16, 16)] = z
        return carry

    lax.fori_loop(0, _K, zrow, 0)


def _zero_acc(rows_v, acc_sh, sid):
    base_r = sid * _RPT8
    for k in range(4):
        pltpu.sync_copy(rows_v, acc_sh.at[pl.ds(base_r + k * 128, 128)])
    pltpu.sync_copy(rows_v.at[pl.ds(0, 112)],
                    acc_sh.at[pl.ds(base_r + 512, 112)])

    @pl.when(sid == _NS - 1)
    def _():
        pltpu.sync_copy(rows_v.at[pl.ds(0, 16)],
                        acc_sh.at[pl.ds(_NS * _RPT8, 16)])


def _write_acc(acc_sh, out, cid, sid):
    pltpu.sync_copy(acc_sh.at[pl.ds(sid * _RPT8, _RPT8)],
                    out.at[pl.ds(cid * _N + sid * _RPT8, _RPT8)])

    @pl.when(sid == _NS - 1)
    def _():
        pltpu.sync_copy(acc_sh.at[pl.ds(_NS * _RPT8, 16)],
                        out.at[pl.ds(cid * _N + _NS * _RPT8, 16)])


def _start_gather(table, gidx, gidx_v, rows_v, sem, c):
    pltpu.sync_copy(gidx.at[pl.ds(c * _K, _K)], gidx_v)
    pltpu.async_copy(table.at[gidx_v], rows_v, sem)


def _segsum_deg_body(table, gidx, sidx, out, dgu, dgi,
                     gidx_v0, gidx_v1, sidx_v0, sidx_v1,
                     rows_v0, rows_v1, ones_v,
                     acc_sh, degu_sh, degi_sh, sem0, sem1):
    cid = lax.axis_index("c")
    sid = lax.axis_index("s")
    wid = sid * _NC + cid
    gv = (gidx_v0, gidx_v1)
    sv = (sidx_v0, sidx_v1)
    rv = (rows_v0, rows_v1)
    sems = (sem0, sem1)

    _zero_rows(rows_v0)
    _zero_acc(rows_v0, acc_sh, sid)
    for k in range(5):
        off = sid * _DPT + k * 128
        pltpu.sync_copy(rows_v0.at[0], degu_sh.at[pl.ds(off, 128)])
        pltpu.sync_copy(rows_v0.at[0], degi_sh.at[pl.ds(off, 128)])
    one = jnp.ones((16,), jnp.float32)
    for c8 in range(8):
        ones_v[pl.ds(c8 * 16, 16)] = one
    plsc.subcore_barrier()

    @pl.when(wid < _CHUNKS)
    def _():
        _start_gather(table, gidx, gv[0], rv[0], sems[0], wid)
        pltpu.sync_copy(sidx.at[pl.ds(wid * _K, _K)], sv[0])

    def trip2(i2, carry):
        for b in range(2):
            i = i2 * 2 + b
            c = wid + i * _NW
            nxt = c + _NW

            @pl.when(nxt < _CHUNKS)
            def _():
                _start_gather(table, gidx, gv[1 - b], rv[1 - b],
                              sems[1 - b], nxt)
                pltpu.sync_copy(sidx.at[pl.ds(nxt * _K, _K)], sv[1 - b])

            @pl.when(c < _CHUNKS)
            def _():
                pltpu.make_async_copy(table.at[gv[b]], rv[b], sems[b]).wait()
                pltpu.sync_copy(rv[b], acc_sh.at[sv[b]], add=True)
                pltpu.sync_copy(ones_v, degu_sh.at[gv[b]], add=True)
                pltpu.sync_copy(ones_v, degi_sh.at[sv[b]], add=True)

        return carry

    lax.fori_loop(0, _TRIPS // 2 + 1, trip2, 0)
    plsc.subcore_barrier()

    _write_acc(acc_sh, out, cid, sid)
    db = cid * _NPAD + sid * _DPT
    pltpu.sync_copy(degu_sh.at[pl.ds(sid * _DPT, _DPT)],
                    dgu.at[pl.ds(db, _DPT)])
    pltpu.sync_copy(degi_sh.at[pl.ds(sid * _DPT, _DPT)],
                    dgi.at[pl.ds(db, _DPT)])


def _segsum_body(table, gidx, sidx, out,
                 gidx_v0, gidx_v1, sidx_v0, sidx_v1,
                 rows_v0, rows_v1, acc_sh, sem0, sem1):
    cid = lax.axis_index("c")
    sid = lax.axis_index("s")
    wid = sid * _NC + cid
    gv = (gidx_v0, gidx_v1)
    sv = (sidx_v0, sidx_v1)
    rv = (rows_v0, rows_v1)
    sems = (sem0, sem1)

    _zero_rows(rows_v0)
    _zero_acc(rows_v0, acc_sh, sid)
    plsc.subcore_barrier()

    @pl.when(wid < _CHUNKS)
    def _():
        _start_gather(table, gidx, gv[0], rv[0], sems[0], wid)
        pltpu.sync_copy(sidx.at[pl.ds(wid * _K, _K)], sv[0])

    def trip2(i2, carry):
        for b in range(2):
            i = i2 * 2 + b
            c = wid + i * _NW
            nxt = c + _NW

            @pl.when(nxt < _CHUNKS)
            def _():
                _start_gather(table, gidx, gv[1 - b], rv[1 - b],
                              sems[1 - b], nxt)
                pltpu.sync_copy(sidx.at[pl.ds(nxt * _K, _K)], sv[1 - b])

            @pl.when(c < _CHUNKS)
            def _():
                pltpu.make_async_copy(table.at[gv[b]], rv[b], sems[b]).wait()
                pltpu.sync_copy(rv[b], acc_sh.at[sv[b]], add=True)

        return carry

    lax.fori_loop(0, _TRIPS // 2 + 1, trip2, 0)
    plsc.subcore_barrier()

    _write_acc(acc_sh, out, cid, sid)


def _dot_body(u_hbm, e_hbm, aidx, bidx, out,
              aidx_v0, aidx_v1, bidx_v0, bidx_v1,
              urows_v0, urows_v1, erows_v0, erows_v1,
              out_v, sem_u0, sem_u1, sem_e0, sem_e1):
    cid = lax.axis_index("c")
    sid = lax.axis_index("s")
    wid = sid * _NC + cid
    av = (aidx_v0, aidx_v1)
    bv = (bidx_v0, bidx_v1)
    uv = (urows_v0, urows_v1)
    ev = (erows_v0, erows_v1)
    su = (sem_u0, sem_u1)
    se = (sem_e0, sem_e1)
    rowiota = lax.iota(jnp.int32, 16)

    @pl.when(wid < _CHUNKS)
    def _():
        _start_gather(u_hbm, aidx, av[0], uv[0], su[0], wid)
        _start_gather(e_hbm, bidx, bv[0], ev[0], se[0], wid)

    def trip2(i2, carry):
        for b in range(2):
            i = i2 * 2 + b
            c = wid + i * _NW
            nxt = c + _NW

            @pl.when(nxt < _CHUNKS)
            def _():
                _start_gather(u_hbm, aidx, av[1 - b], uv[1 - b],
                              su[1 - b], nxt)
                _start_gather(e_hbm, bidx, bv[1 - b], ev[1 - b],
                              se[1 - b], nxt)

            @pl.when(c < _CHUNKS)
            def _():
                urows_v = uv[b]
                erows_v = ev[b]
                pltpu.make_async_copy(u_hbm.at[av[b]], urows_v, su[b]).wait()
                pltpu.make_async_copy(e_hbm.at[bv[b]], erows_v, se[b]).wait()

                def gloop(g, carry):
                    gbase = g * 16
                    out_vec = _Z16()
                    for e in range(16):
                        row = gbase + e
                        acc = _Z16()
                        for k8 in range(8):
                            uc = urows_v[row, pl.ds(k8 * 16, 16)]
                            ec = erows_v[row, pl.ds(k8 * 16, 16)]
                            acc = acc + uc * ec
                        for sh in (8, 4, 2, 1):
                            pidx = (rowiota + sh) & 15
                            acc = acc + acc.at[pidx].get(
                                mode="promise_in_bounds")
                        out_vec = jnp.where(rowiota == e, acc, out_vec)
                    out_v[pl.ds(gbase, 16)] = out_vec
                    return carry

                lax.fori_loop(0, 8, gloop, 0)
                pltpu.sync_copy(out_v, out.at[pl.ds(c * _K, _K)])

        return carry

    lax.fori_loop(0, _TRIPS // 2 + 1, trip2, 0)


_segsum_deg = pl.kernel(
    _segsum_deg_body,
    out_type=(jax.ShapeDtypeStruct((_NC * _N, _D), jnp.float32),
              jax.ShapeDtypeStruct((_NC * _NPAD,), jnp.float32),
              jax.ShapeDtypeStruct((_NC * _NPAD,), jnp.float32)),
    mesh=_mesh,
    scratch_types=(pltpu.VMEM((_K,), jnp.int32),
                   pltpu.VMEM((_K,), jnp.int32),
                   pltpu.VMEM((_K,), jnp.int32),
                   pltpu.VMEM((_K,), jnp.int32),
                   pltpu.VMEM((_K, _D), jnp.float32),
                   pltpu.VMEM((_K, _D), jnp.float32),
                   pltpu.VMEM((_K,), jnp.float32),
                   pltpu.VMEM_SHARED((_N, _D), jnp.float32),
                   pltpu.VMEM_SHARED((_NPAD,), jnp.float32),
                   pltpu.VMEM_SHARED((_NPAD,), jnp.float32),
                   pltpu.SemaphoreType.DMA,
                   pltpu.SemaphoreType.DMA),
)

_segsum = pl.kernel(
    _segsum_body,
    out_type=jax.ShapeDtypeStruct((_NC * _N, _D), jnp.float32),
    mesh=_mesh,
    scratch_types=(pltpu.VMEM((_K,), jnp.int32),
                   pltpu.VMEM((_K,), jnp.int32),
                   pltpu.VMEM((_K,), jnp.int32),
                   pltpu.VMEM((_K,), jnp.int32),
                   pltpu.VMEM((_K, _D), jnp.float32),
                   pltpu.VMEM((_K, _D), jnp.float32),
                   pltpu.VMEM_SHARED((_N, _D), jnp.float32),
                   pltpu.SemaphoreType.DMA,
                   pltpu.SemaphoreType.DMA),
)

_dot = pl.kernel(
    _dot_body,
    out_type=jax.ShapeDtypeStruct((_E,), jnp.float32),
    mesh=_mesh,
    scratch_types=(pltpu.VMEM((_K,), jnp.int32),
                   pltpu.VMEM((_K,), jnp.int32),
                   pltpu.VMEM((_K,), jnp.int32),
                   pltpu.VMEM((_K,), jnp.int32),
                   pltpu.VMEM((_K, _D), jnp.float32),
                   pltpu.VMEM((_K, _D), jnp.float32),
                   pltpu.VMEM((_K, _D), jnp.float32),
                   pltpu.VMEM((_K, _D), jnp.float32),
                   pltpu.VMEM((_K,), jnp.float32),
                   pltpu.SemaphoreType.DMA,
                   pltpu.SemaphoreType.DMA,
                   pltpu.SemaphoreType.DMA,
                   pltpu.SemaphoreType.DMA),
)


def _norm_items_fn(p_ref, deg_ref, o_ref):
    inv = lax.rsqrt(deg_ref[0] + deg_ref[1])
    o_ref[:] = (p_ref[0] + p_ref[1]) * inv


def _norm_users_fn(q_ref, deg_ref, u0_ref, u2_ref, u_ref):
    inv = lax.rsqrt(deg_ref[0] + deg_ref[1])
    u2 = (q_ref[0] + q_ref[1]) * inv
    u2_ref[:] = u2
    u_ref[:] = 0.5 * (u0_ref[:] + u2)


def _norm_items2_fn(r_ref, deg_ref, e1_ref, e_ref):
    inv = lax.rsqrt(deg_ref[0] + deg_ref[1])
    e_ref[:] = 0.5 * (e1_ref[:] + (r_ref[0] + r_ref[1]) * inv)


_p_spec = pl.BlockSpec((2, _BR, _D), lambda i: (0, i, 0))
_deg_spec = pl.BlockSpec((2, _BR, 1), lambda i: (0, i, 0))
_row_spec = pl.BlockSpec((_BR, _D), lambda i: (i, 0))

_norm_items = pl.pallas_call(
    _norm_items_fn,
    out_shape=jax.ShapeDtypeStruct((_N, _D), jnp.float32),
    grid=(_N // _BR,),
    in_specs=[_p_spec, _deg_spec],
    out_specs=_row_spec,
)

_norm_users = pl.pallas_call(
    _norm_users_fn,
    out_shape=(jax.ShapeDtypeStruct((_N, _D), jnp.float32),
               jax.ShapeDtypeStruct((_N, _D), jnp.float32)),
    grid=(_N // _BR,),
    in_specs=[_p_spec, _deg_spec, _row_spec],
    out_specs=(_row_spec, _row_spec),
)

_norm_items2 = pl.pallas_call(
    _norm_items2_fn,
    out_shape=jax.ShapeDtypeStruct((_N, _D), jnp.float32),
    grid=(_N // _BR,),
    in_specs=[_p_spec, _deg_spec, _row_spec],
    out_specs=_row_spec,
)


def kernel(edge_index, edge_label_index, n_items, U_0):
    src = edge_index[0]
    dst = (edge_index[1] % n_items).astype(jnp.int32)
    a = edge_label_index[0]
    b = edge_label_index[1]

    p1, dgu, dgi = _segsum_deg(U_0, src, dst)
    dgu = dgu.reshape(_NC, _NPAD)[:, :_N, None]
    dgi = dgi.reshape(_NC, _NPAD)[:, :_N, None]
    E1 = _norm_items(p1.reshape(_NC, _N, _D), dgi)
    q = _segsum(E1, dst, src)
    U2, U = _norm_users(q.reshape(_NC, _N, _D), dgu, U_0)
    r = _segsum(U2, src, dst)
    E = _norm_items2(r.reshape(_NC, _N, _D), dgi, E1)
    return _dot(U, E, a, b)


# speedup vs baseline: 8.5379x; 1.4834x over previous
import functools

import jax
import jax.numpy as jnp
from jax import lax
from jax.experimental import pallas as pl
from jax.experimental.pallas import tpu as pltpu
from jax.experimental.pallas import tpu_sc as plsc

_N = 10000
_E = 320000
_D = 128
_NC = 2
_NS = 16
_NW = _NC * _NS
_K = 128
_CHUNKS = _E // _K
_CPW = 80
_CHUNKS_PAD = _NW * _CPW
_IB = 8
_NB = _CPW // _IB
_RPT8 = 624
_NPAD = 10240
_DPT = _NPAD // _NS
_BR = 1000

_mesh = plsc.VectorSubcoreMesh(core_axis_name="c", subcore_axis_name="s")

_Z16 = functools.partial(jnp.zeros, (16,), jnp.float32)


def _zero_rows(rows_v):
    z = _Z16()

    def zrow(r, carry):
        for c8 in range(8):
            rows_v[r, pl.ds(c8 * 16, 16)] = z
        return carry

    lax.fori_loop(0, _K, zrow, 0)


def _zero_acc(rows_v, acc_sh, sid):
    base_r = sid * _RPT8
    for k in range(4):
        pltpu.sync_copy(rows_v, acc_sh.at[pl.ds(base_r + k * 128, 128)])
    pltpu.sync_copy(rows_v.at[pl.ds(0, 112)],
                    acc_sh.at[pl.ds(base_r + 512, 112)])

    @pl.when(sid == _NS - 1)
    def _():
        pltpu.sync_copy(rows_v.at[pl.ds(0, 16)],
                        acc_sh.at[pl.ds(_NS * _RPT8, 16)])


def _write_acc(acc_sh, out, cid, sid):
    pltpu.sync_copy(acc_sh.at[pl.ds(sid * _RPT8, _RPT8)],
                    out.at[pl.ds(cid * _N + sid * _RPT8, _RPT8)])

    @pl.when(sid == _NS - 1)
    def _():
        pltpu.sync_copy(acc_sh.at[pl.ds(_NS * _RPT8, 16)],
                        out.at[pl.ds(cid * _N + _NS * _RPT8, 16)])


def _segsum_generic(with_deg, table, gidx, sidx, out, dgu, dgi,
                    gb0, gb1, sb0, sb1, rows_v0, rows_v1, ones_v,
                    acc_sh, degu_sh, degi_sh,
                    gsem0, gsem1, ssem0, ssem1):
    cid = lax.axis_index("c")
    sid = lax.axis_index("s")
    wid = sid * _NC + cid
    wstart = wid * _CPW
    gb = (gb0, gb1)
    sb = (sb0, sb1)
    rv = (rows_v0, rows_v1)
    gsem = (gsem0, gsem1)
    ssem = (ssem0, ssem1)

    _zero_rows(rows_v0)
    _zero_acc(rows_v0, acc_sh, sid)
    if with_deg:
        for k in range(5):
            off = sid * _DPT + k * 128
            pltpu.sync_copy(rows_v0.at[0], degu_sh.at[pl.ds(off, 128)])
            pltpu.sync_copy(rows_v0.at[0], degi_sh.at[pl.ds(off, 128)])
        one = jnp.ones((16,), jnp.float32)
        for c8 in range(8):
            ones_v[pl.ds(c8 * 16, 16)] = one
    plsc.subcore_barrier()

    pltpu.sync_copy(gidx.at[pl.ds(wstart, _IB)], gb[0])
    pltpu.sync_copy(sidx.at[pl.ds(wstart, _IB)], sb[0])

    @pl.when(wstart < _CHUNKS)
    def _():
        pltpu.async_copy(table.at[gb[0].at[0]], rv[0], gsem[0])

    def batch2(j2, carry):
      for jb in range(2):
        j = j2 * 2 + jb
        nb = 1 - jb
        bstart = wstart + (j + 1) * _IB

        @pl.when(bstart < _CHUNKS)
        def _():
            pltpu.sync_copy(gidx.at[pl.ds(bstart, _IB)], gb[nb])
            pltpu.sync_copy(sidx.at[pl.ds(bstart, _IB)], sb[nb])

        for t in range(_IB):
            i = j * _IB + t
            c = wstart + i
            p = t % 2
            n_buf = gb[jb] if t < _IB - 1 else gb[nb]
            n_row = t + 1 if t < _IB - 1 else 0
            pre_ok = (c + 1 < _CHUNKS) if t < _IB - 1 else (
                (j < _NB - 1) & (c + 1 < _CHUNKS))

            @pl.when(pre_ok)
            def _():
                @pl.when(i > 0)
                def _():
                    pltpu.make_async_copy(
                        rv[1 - p], acc_sh.at[sb[jb].at[0]],
                        ssem[1 - p]).wait()

                pltpu.async_copy(table.at[n_buf.at[n_row]], rv[1 - p],
                                 gsem[1 - p])

            @pl.when(c < _CHUNKS)
            def _():
                pltpu.make_async_copy(table.at[gb[jb].at[t]], rv[p],
                                      gsem[p]).wait()
                pltpu.async_copy(rv[p], acc_sh.at[sb[jb].at[t]], ssem[p],
                                 add=True)
                if with_deg:
                    pltpu.sync_copy(ones_v, degu_sh.at[gb[jb].at[t]],
                                    add=True)
                    pltpu.sync_copy(ones_v, degi_sh.at[sb[jb].at[t]],
                                    add=True)

      return carry

    lax.fori_loop(0, _NB // 2, batch2, 0)
    pltpu.make_async_copy(rv[0], acc_sh.at[sb[0].at[0]], ssem[0]).wait()
    pltpu.make_async_copy(rv[1], acc_sh.at[sb[0].at[0]], ssem[1]).wait()

    plsc.subcore_barrier()

    _write_acc(acc_sh, out, cid, sid)
    if with_deg:
        db = cid * _NPAD + sid * _DPT
        pltpu.sync_copy(degu_sh.at[pl.ds(sid * _DPT, _DPT)],
                        dgu.at[pl.ds(db, _DPT)])
        pltpu.sync_copy(degi_sh.at[pl.ds(sid * _DPT, _DPT)],
                        dgi.at[pl.ds(db, _DPT)])


def _segsum_deg_body(table, gidx, sidx, out, dgu, dgi,
                     gb0, gb1, sb0, sb1, rows_v0, rows_v1, ones_v,
                     acc_sh, degu_sh, degi_sh,
                     gsem0, gsem1, ssem0, ssem1):
    _segsum_generic(True, table, gidx, sidx, out, dgu, dgi,
                    gb0, gb1, sb0, sb1, rows_v0, rows_v1, ones_v,
                    acc_sh, degu_sh, degi_sh,
                    gsem0, gsem1, ssem0, ssem1)


def _segsum_body(table, gidx, sidx, out,
                 gb0, gb1, sb0, sb1, rows_v0, rows_v1,
                 acc_sh, gsem0, gsem1, ssem0, ssem1):
    _segsum_generic(False, table, gidx, sidx, out, None, None,
                    gb0, gb1, sb0, sb1, rows_v0, rows_v1, None,
                    acc_sh, None, None,
                    gsem0, gsem1, ssem0, ssem1)


def _dot_body(u_hbm, e_hbm, aidx, bidx, out,
              ab0, ab1, bb0, bb1,
              urows_v0, urows_v1, erows_v0, erows_v1,
              out_v, sem_u0, sem_u1, sem_e0, sem_e1):
    cid = lax.axis_index("c")
    sid = lax.axis_index("s")
    wid = sid * _NC + cid
    wstart = wid * _CPW
    ab = (ab0, ab1)
    bb = (bb0, bb1)
    uv = (urows_v0, urows_v1)
    ev = (erows_v0, erows_v1)
    su = (sem_u0, sem_u1)
    se = (sem_e0, sem_e1)
    rowiota = lax.iota(jnp.int32, 16)

    pltpu.sync_copy(aidx.at[pl.ds(wstart, _IB)], ab[0])
    pltpu.sync_copy(bidx.at[pl.ds(wstart, _IB)], bb[0])

    @pl.when(wstart < _CHUNKS)
    def _():
        pltpu.async_copy(u_hbm.at[ab[0].at[0]], uv[0], su[0])
        pltpu.async_copy(e_hbm.at[bb[0].at[0]], ev[0], se[0])

    def batch2(j2, carry):
      for jb in range(2):
        j = j2 * 2 + jb
        nb = 1 - jb
        bstart = wstart + (j + 1) * _IB

        @pl.when(bstart < _CHUNKS)
        def _():
            pltpu.sync_copy(aidx.at[pl.ds(bstart, _IB)], ab[nb])
            pltpu.sync_copy(bidx.at[pl.ds(bstart, _IB)], bb[nb])

        for t in range(_IB):
            i = j * _IB + t
            c = wstart + i
            p = t % 2
            na_buf = ab[jb] if t < _IB - 1 else ab[nb]
            nb_buf = bb[jb] if t < _IB - 1 else bb[nb]
            n_row = t + 1 if t < _IB - 1 else 0
            pre_ok = (c + 1 < _CHUNKS) if t < _IB - 1 else (
                (j < _NB - 1) & (c + 1 < _CHUNKS))

            @pl.when(pre_ok)
            def _():
                pltpu.async_copy(u_hbm.at[na_buf.at[n_row]], uv[1 - p],
                                 su[1 - p])
                pltpu.async_copy(e_hbm.at[nb_buf.at[n_row]], ev[1 - p],
                                 se[1 - p])

            @pl.when(c < _CHUNKS)
            def _():
                urows_v = uv[p]
                erows_v = ev[p]
                pltpu.make_async_copy(u_hbm.at[ab[jb].at[t]], urows_v,
                                      su[p]).wait()
                pltpu.make_async_copy(e_hbm.at[bb[jb].at[t]], erows_v,
                                      se[p]).wait()

                def gloop(g, carry):
                    gbase = g * 16

                    def eloop(e2, out_vec):
                        for sub in range(2):
                            e = e2 * 2 + sub
                            row = gbase + e
                            acc = _Z16()
                            for k8 in range(8):
                                uc = urows_v[row, pl.ds(k8 * 16, 16)]
                                ec = erows_v[row, pl.ds(k8 * 16, 16)]
                                acc = acc + uc * ec
                            for sh in (8, 4, 2, 1):
                                pidx = (rowiota + sh) & 15
                                acc = acc + acc.at[pidx].get(
                                    mode="promise_in_bounds")
                            out_vec = jnp.where(rowiota == e, acc, out_vec)
                        return out_vec

                    out_vec = lax.fori_loop(0, 8, eloop, _Z16())
                    out_v[pl.ds(gbase, 16)] = out_vec
                    return carry

                lax.fori_loop(0, 8, gloop, 0)
                pltpu.sync_copy(out_v, out.at[pl.ds(c * _K, _K)])

      return carry

    lax.fori_loop(0, _NB // 2, batch2, 0)


_idx2d = jax.ShapeDtypeStruct((_CHUNKS_PAD, _K), jnp.int32)

_segsum_deg = pl.kernel(
    _segsum_deg_body,
    out_type=(jax.ShapeDtypeStruct((_NC * _N, _D), jnp.float32),
              jax.ShapeDtypeStruct((_NC * _NPAD,), jnp.float32),
              jax.ShapeDtypeStruct((_NC * _NPAD,), jnp.float32)),
    mesh=_mesh,
    scratch_types=(pltpu.VMEM((_IB, _K), jnp.int32),
                   pltpu.VMEM((_IB, _K), jnp.int32),
                   pltpu.VMEM((_IB, _K), jnp.int32),
                   pltpu.VMEM((_IB, _K), jnp.int32),
                   pltpu.VMEM((_K, _D), jnp.float32),
                   pltpu.VMEM((_K, _D), jnp.float32),
                   pltpu.VMEM((_K,), jnp.float32),
                   pltpu.VMEM_SHARED((_N, _D), jnp.float32),
                   pltpu.VMEM_SHARED((_NPAD,), jnp.float32),
                   pltpu.VMEM_SHARED((_NPAD,), jnp.float32),
                   pltpu.SemaphoreType.DMA,
                   pltpu.SemaphoreType.DMA,
                   pltpu.SemaphoreType.DMA,
                   pltpu.SemaphoreType.DMA),
)

_segsum = pl.kernel(
    _segsum_body,
    out_type=jax.ShapeDtypeStruct((_NC * _N, _D), jnp.float32),
    mesh=_mesh,
    scratch_types=(pltpu.VMEM((_IB, _K), jnp.int32),
                   pltpu.VMEM((_IB, _K), jnp.int32),
                   pltpu.VMEM((_IB, _K), jnp.int32),
                   pltpu.VMEM((_IB, _K), jnp.int32),
                   pltpu.VMEM((_K, _D), jnp.float32),
                   pltpu.VMEM((_K, _D), jnp.float32),
                   pltpu.VMEM_SHARED((_N, _D), jnp.float32),
                   pltpu.SemaphoreType.DMA,
                   pltpu.SemaphoreType.DMA,
                   pltpu.SemaphoreType.DMA,
                   pltpu.SemaphoreType.DMA),
)

_dot = pl.kernel(
    _dot_body,
    out_type=jax.ShapeDtypeStruct((_E,), jnp.float32),
    mesh=_mesh,
    scratch_types=(pltpu.VMEM((_IB, _K), jnp.int32),
                   pltpu.VMEM((_IB, _K), jnp.int32),
                   pltpu.VMEM((_IB, _K), jnp.int32),
                   pltpu.VMEM((_IB, _K), jnp.int32),
                   pltpu.VMEM((_K, _D), jnp.float32),
                   pltpu.VMEM((_K, _D), jnp.float32),
                   pltpu.VMEM((_K, _D), jnp.float32),
                   pltpu.VMEM((_K, _D), jnp.float32),
                   pltpu.VMEM((_K,), jnp.float32),
                   pltpu.SemaphoreType.DMA,
                   pltpu.SemaphoreType.DMA,
                   pltpu.SemaphoreType.DMA,
                   pltpu.SemaphoreType.DMA),
)


def _norm_items_fn(p_ref, deg_ref, o_ref):
    inv = lax.rsqrt(deg_ref[0] + deg_ref[1])
    o_ref[:] = (p_ref[0] + p_ref[1]) * inv


def _norm_users_fn(q_ref, deg_ref, u0_ref, u2_ref, u_ref):
    inv = lax.rsqrt(deg_ref[0] + deg_ref[1])
    u2 = (q_ref[0] + q_ref[1]) * inv
    u2_ref[:] = u2
    u_ref[:] = 0.5 * (u0_ref[:] + u2)


def _norm_items2_fn(r_ref, deg_ref, e1_ref, e_ref):
    inv = lax.rsqrt(deg_ref[0] + deg_ref[1])
    e_ref[:] = 0.5 * (e1_ref[:] + (r_ref[0] + r_ref[1]) * inv)


_p_spec = pl.BlockSpec((2, _BR, _D), lambda i: (0, i, 0))
_deg_spec = pl.BlockSpec((2, _BR, 1), lambda i: (0, i, 0))
_row_spec = pl.BlockSpec((_BR, _D), lambda i: (i, 0))

_norm_items = pl.pallas_call(
    _norm_items_fn,
    out_shape=jax.ShapeDtypeStruct((_N, _D), jnp.float32),
    grid=(_N // _BR,),
    in_specs=[_p_spec, _deg_spec],
    out_specs=_row_spec,
)

_norm_users = pl.pallas_call(
    _norm_users_fn,
    out_shape=(jax.ShapeDtypeStruct((_N, _D), jnp.float32),
               jax.ShapeDtypeStruct((_N, _D), jnp.float32)),
    grid=(_N // _BR,),
    in_specs=[_p_spec, _deg_spec, _row_spec],
    out_specs=(_row_spec, _row_spec),
)

_norm_items2 = pl.pallas_call(
    _norm_items2_fn,
    out_shape=jax.ShapeDtypeStruct((_N, _D), jnp.float32),
    grid=(_N // _BR,),
    in_specs=[_p_spec, _deg_spec, _row_spec],
    out_specs=_row_spec,
)


def _pad_idx(v):
    return jnp.pad(v, (0, _CHUNKS_PAD * _K - _E)).reshape(_CHUNKS_PAD, _K)


def kernel(edge_index, edge_label_index, n_items, U_0):
    src = _pad_idx(edge_index[0])
    dst = _pad_idx((edge_index[1] % n_items).astype(jnp.int32))
    a = _pad_idx(edge_label_index[0])
    b = _pad_idx(edge_label_index[1])

    p1, dgu, dgi = _segsum_deg(U_0, src, dst)
    dgu = dgu.reshape(_NC, _NPAD)[:, :_N, None]
    dgi = dgi.reshape(_NC, _NPAD)[:, :_N, None]
    E1 = _norm_items(p1.reshape(_NC, _N, _D), dgi)
    q = _segsum(E1, dst, src)
    U2, U = _norm_users(q.reshape(_NC, _N, _D), dgu, U_0)
    r = _segsum(U2, src, dst)
    E = _norm_items2(r.reshape(_NC, _N, _D), dgi, E1)
    return _dot(U, E, a, b)
